# baseline (speedup 1.0000x reference)
import functools

import jax
import jax.numpy as jnp
from jax import lax
from jax.experimental import pallas as pl
from jax.experimental.pallas import tpu as pltpu
from jax.experimental.pallas import tpu_sc as plsc

_N = 10000
_E = 160000
_NP = 10240
_EP = 163840
_NC = 2
_NS = 16
_NW = _NC * _NS
_CHUNK = 128

_ROWS_PER_TILE = _NP // _NS


def _sc_gather(table, idx):
    d = table.shape[1]
    per_w = _EP // _NW
    n_chunks = per_w // _CHUNK
    mesh = plsc.VectorSubcoreMesh(core_axis_name="c", subcore_axis_name="s")

    @functools.partial(
        pl.kernel, mesh=mesh,
        out_type=jax.ShapeDtypeStruct((_EP, d), jnp.float32),
        scratch_types=[
            pltpu.VMEM((_CHUNK,), jnp.int32),
            pltpu.VMEM((_CHUNK, d), jnp.float32),
            pltpu.SemaphoreType.DMA,
        ],
    )
    def k(table_hbm, idx_hbm, out_hbm, idx_v, rows_v, sem):
        wid = lax.axis_index("s") * _NC + lax.axis_index("c")
        base = wid * per_w

        def body(i, carry):
            off = base + i * _CHUNK
            pltpu.sync_copy(idx_hbm.at[pl.ds(off, _CHUNK)], idx_v)
            pltpu.async_copy(table_hbm.at[idx_v], rows_v, sem).wait()
            pltpu.sync_copy(rows_v, out_hbm.at[pl.ds(off, _CHUNK)])
            return carry

        lax.fori_loop(0, n_chunks, body, 0)

    return k(table, idx)


def _sc_scatter_add(msg, idx, zeros_nd):
    d = msg.shape[1]
    per_w = _EP // _NW
    n_chunks = per_w // _CHUNK
    mesh = plsc.VectorSubcoreMesh(core_axis_name="c", subcore_axis_name="s")

    @functools.partial(
        pl.kernel, mesh=mesh,
        out_type=jax.ShapeDtypeStruct((_NC, _NP, d), jnp.float32),
        scratch_types=[
            pltpu.VMEM((_CHUNK,), jnp.int32),
            pltpu.VMEM((_CHUNK, d), jnp.float32),
            pltpu.VMEM_SHARED((_NP, d), jnp.float32),
        ],
    )
    def k(msg_hbm, idx_hbm, zero_hbm, out_hbm, idx_v, chunk_v, acc_sh):
        cid = lax.axis_index("c")
        sid = lax.axis_index("s")
        wid = sid * _NC + cid
        r0 = sid * _ROWS_PER_TILE
        pltpu.sync_copy(zero_hbm.at[pl.ds(r0, _ROWS_PER_TILE)],
                        acc_sh.at[pl.ds(r0, _ROWS_PER_TILE)])
        plsc.subcore_barrier()
        base = wid * per_w

        def body(i, carry):
            off = base + i * _CHUNK
            pltpu.sync_copy(idx_hbm.at[pl.ds(off, _CHUNK)], idx_v)
            pltpu.sync_copy(msg_hbm.at[pl.ds(off, _CHUNK)], chunk_v)
            pltpu.sync_copy(chunk_v, acc_sh.at[idx_v], add=True)
            return carry

        lax.fori_loop(0, n_chunks, body, 0)
        plsc.subcore_barrier()
        pltpu.sync_copy(acc_sh.at[pl.ds(r0, _ROWS_PER_TILE)],
                        out_hbm.at[cid, pl.ds(r0, _ROWS_PER_TILE)])

    return k(msg, idx, zeros_nd)


def _ln(h, g, b):
    m = jnp.mean(h, axis=-1, keepdims=True)
    v = jnp.mean((h - m) ** 2, axis=-1, keepdims=True)
    return (h - m) * lax.rsqrt(v + 1e-5) * g + b


def _full2d(arr):
    return pl.BlockSpec(arr.shape, lambda i: (0, 0))


_NODE_T = 1024


def _rowspec(width, tile=_NODE_T):
    return pl.BlockSpec((tile, width), lambda i: (i, 0))


def _embed(xp, w0, b0, g0, be0, w1, b1, g1, be1):
    def body(x_ref, w0r, b0r, g0r, be0r, w1r, b1r, g1r, be1r, out_ref):
        h = jnp.maximum(x_ref[...] @ w0r[...] + b0r[...], 0.0)
        h = _ln(h, g0r[...], be0r[...])
        h = jnp.maximum(h @ w1r[...] + b1r[...], 0.0)
        out_ref[...] = _ln(h, g1r[...], be1r[...])

    ws = [w0, b0, g0, be0, w1, b1, g1, be1]
    return pl.pallas_call(
        body,
        grid=(_NP // _NODE_T,),
        in_specs=[_rowspec(9)] + [_full2d(w) for w in ws],
        out_specs=_rowspec(64),
        out_shape=jax.ShapeDtypeStruct((_NP, 64), jnp.float32),
    )(xp, *ws)


_EDGE_T = 512


def _msg(ea, xs, w1, b1, w2p, b2r, out_c, in_c, out_w):
    xs_w = xs.shape[1]

    def body(ea_ref, xs_ref, w1r, b1r, w2pr, b2rr, out_ref):
        hh = jnp.maximum(ea_ref[...] @ w1r[...] + b1r[...], 0.0)
        xsv = xs_ref[...][:, :in_c]
        p = xsv @ w2pr[...]
        acc = xsv @ b2rr[...]
        for k in range(32):
            acc = acc + hh[:, k:k + 1] * p[:, k * out_c:(k + 1) * out_c]
        if out_w > out_c:
            acc = jnp.concatenate(
                [acc, jnp.zeros((acc.shape[0], out_w - out_c), jnp.float32)],
                axis=1)
        out_ref[...] = acc

    ws = [w1, b1, w2p, b2r]
    return pl.pallas_call(
        body,
        grid=(_EP // _EDGE_T,),
        in_specs=[_rowspec(8, _EDGE_T), _rowspec(xs_w, _EDGE_T)]
        + [_full2d(w) for w in ws],
        out_specs=_rowspec(out_w, _EDGE_T),
        out_shape=jax.ShapeDtypeStruct((_EP, out_w), jnp.float32),
    )(ea, xs, *ws)


def _node_update(h, a0, a1, root, bias, g, be):
    in_c = h.shape[1]
    out_c = root.shape[1]
    agg_w = a0.shape[1]

    def body(h_ref, a0_ref, a1_ref, rootr, biasr, gr, ber, out_ref):
        aggr = a0_ref[...][:, :out_c] + a1_ref[...][:, :out_c]
        s = aggr + h_ref[...] @ rootr[...] + biasr[...]
        out_ref[...] = _ln(jnp.maximum(s, 0.0), gr[...], ber[...])

    ws = [root, bias, g, be]
    return pl.pallas_call(
        body,
        grid=(_NP // _NODE_T,),
        in_specs=[_rowspec(in_c), _rowspec(agg_w), _rowspec(agg_w)]
        + [_full2d(w) for w in ws],
        out_specs=_rowspec(out_c),
        out_shape=jax.ShapeDtypeStruct((_NP, out_c), jnp.float32),
    )(h, a0, a1, *ws)


def _qkv_proj(h, wqkv, bqkv):
    def body(h_ref, wr, br, out_ref):
        out_ref[...] = h_ref[...] @ wr[...] + br[...]

    return pl.pallas_call(
        body,
        grid=(_NP // _NODE_T,),
        in_specs=[_rowspec(32), _full2d(wqkv), _full2d(bqkv)],
        out_specs=_rowspec(96),
        out_shape=jax.ShapeDtypeStruct((_NP, 96), jnp.float32),
    )(h, wqkv, bqkv)


_ATT_TQ = 256
_ATT_TK = 1024
_NHEAD = 4
_HD = 8


def _enc_layer(h, qkv, wo, bo, g1, be1, w1, b1, w2, b2, g2, be2):
    scale = 1.0 / (_HD ** 0.5)
    n_kt = _NP // _ATT_TK

    def body(h_ref, qkv_ref, wor, bor, g1r, be1r, w1r, b1r, w2r, b2r,
             g2r, be2r, out_ref):
        i = pl.program_id(0)
        q_all = qkv_ref[pl.ds(i * _ATT_TQ, _ATT_TQ), 0:32] * scale
        outs = []
        for hd in range(_NHEAD):
            qh = q_all[:, hd * _HD:(hd + 1) * _HD]

            def kv_step(j, carry, hd=hd, qh=qh):
                m, l, acc = carry
                kv = qkv_ref[pl.ds(j * _ATT_TK, _ATT_TK), :]
                kh = kv[:, 32 + hd * _HD:32 + (hd + 1) * _HD]
                vh = kv[:, 64 + hd * _HD:64 + (hd + 1) * _HD]
                s = lax.dot_general(qh, kh, (((1,), (1,)), ((), ())))
                kidx = j * _ATT_TK + lax.broadcasted_iota(
                    jnp.int32, (_ATT_TQ, _ATT_TK), 1)
                s = jnp.where(kidx >= _N, -1e30, s)
                m_new = jnp.maximum(m, jnp.max(s, axis=-1, keepdims=True))
                pexp = jnp.exp(s - m_new)
                corr = jnp.exp(m - m_new)
                l_new = l * corr + jnp.sum(pexp, axis=-1, keepdims=True)
                acc_new = acc * corr + pexp @ vh
                return m_new, l_new, acc_new

            m0 = jnp.full((_ATT_TQ, 1), -1e30, jnp.float32)
            l0 = jnp.zeros((_ATT_TQ, 1), jnp.float32)
            a0 = jnp.zeros((_ATT_TQ, _HD), jnp.float32)
            m, l, acc = lax.fori_loop(0, n_kt, kv_step, (m0, l0, a0))
            outs.append(acc / l)
        o = jnp.concatenate(outs, axis=-1)
        attn = o @ wor[...] + bor[...]
        x1 = _ln(h_ref[...] + attn, g1r[...], be1r[...])
        f = jnp.maximum(x1 @ w1r[...] + b1r[...], 0.0) @ w2r[...] + b2r[...]
        out_ref[...] = _ln(x1 + f, g2r[...], be2r[...])

    ws = [wo, bo, g1, be1, w1, b1, w2, b2, g2, be2]
    return pl.pallas_call(
        body,
        grid=(_NP // _ATT_TQ,),
        in_specs=[_rowspec(32, _ATT_TQ), _full2d(qkv)]
        + [_full2d(w) for w in ws],
        out_specs=_rowspec(32, _ATT_TQ),
        out_shape=jax.ShapeDtypeStruct((_NP, 32), jnp.float32),
    )(h, qkv, *ws)


def _head(h, fc_w, fc_b, out_w, out_b):
    def body(h_ref, fwr, fbr, owr, obr, out_ref):
        f = jnp.maximum(h_ref[...] @ fwr[...] + fbr[...], 0.0)
        out_ref[...] = f @ owr[...] + obr[...]

    ws = [fc_w, fc_b, out_w, out_b]
    return pl.pallas_call(
        body,
        grid=(_NP // _NODE_T,),
        in_specs=[_rowspec(32)] + [_full2d(w) for w in ws],
        out_specs=_rowspec(7),
        out_shape=jax.ShapeDtypeStruct((_NP, 7), jnp.float32),
    )(h, *ws)


def _r2(v):
    return v.reshape(1, -1)


def kernel(x, edge_index, edge_attr, params):
    p = params
    f32 = jnp.float32
    src = edge_index[0]
    dst = edge_index[1]

    xp = jnp.zeros((_NP, 9), f32).at[:_N].set(x)
    eap = jnp.zeros((_EP, 8), f32).at[:_E].set(edge_attr)
    srcp = jnp.zeros((_EP,), jnp.int32).at[:_E].set(src)
    dstp = jnp.full((_EP,), _N, jnp.int32).at[:_E].set(dst)
    z128 = jnp.zeros((_NP, 128), f32)

    w2p1 = p['c1_w2'].reshape(32, 64, 128).transpose(1, 0, 2).reshape(64, 32 * 128)
    b2r1 = p['c1_b2'].reshape(64, 128)
    w2p2 = p['c2_w2'].reshape(32, 128, 32).transpose(1, 0, 2).reshape(128, 32 * 32)
    b2r2 = p['c2_b2'].reshape(128, 32)

    h64 = _embed(xp, p['emb_w0'], _r2(p['emb_b0']), _r2(p['emb_g0']),
                 _r2(p['emb_be0']), p['emb_w1'], _r2(p['emb_b1']),
                 _r2(p['emb_g1']), _r2(p['emb_be1']))

    h64p = jnp.concatenate([h64, jnp.zeros((_NP, 64), f32)], axis=1)
    xs1 = _sc_gather(h64p, srcp)
    msg1 = _msg(eap, xs1, p['c1_w1'], _r2(p['c1_b1']), w2p1, b2r1, 128, 64, 128)
    agg1 = _sc_scatter_add(msg1, dstp, z128)
    h128 = _node_update(h64, agg1[0], agg1[1], p['c1_root'],
                        _r2(p['c1_bias']), _r2(p['n1_g']), _r2(p['n1_be']))

    xs2 = _sc_gather(h128, srcp)
    msg2 = _msg(eap, xs2, p['c2_w1'], _r2(p['c2_b1']), w2p2, b2r2, 32, 128, 128)
    agg2 = _sc_scatter_add(msg2, dstp, z128)
    h32 = _node_update(h128, agg2[0], agg2[1], p['c2_root'],
                       _r2(p['c2_bias']), _r2(p['n2_g']), _r2(p['n2_be']))

    for lp in p['layers']:
        qkv = _qkv_proj(h32, lp['wqkv'], _r2(lp['bqkv']))
        h32 = _enc_layer(h32, qkv, lp['wo'], _r2(lp['bo']), _r2(lp['g1']),
                         _r2(lp['be1']), lp['w1'], _r2(lp['b1']), lp['w2'],
                         _r2(lp['b2']), _r2(lp['g2']), _r2(lp['be2']))

    out = _head(h32, p['fc_w'], _r2(p['fc_b']), p['out_w'], _r2(p['out_b']))
    return out[:_N]


# speedup vs baseline: 1.2358x; 1.2358x over previous
import functools

import jax
import jax.numpy as jnp
from jax import lax
from jax.experimental import pallas as pl
from jax.experimental.pallas import tpu as pltpu
from jax.experimental.pallas import tpu_sc as plsc

_N = 10000
_E = 160000
_NP = 10240
_EP = 163840
_NC = 2
_NS = 16
_NW = _NC * _NS
_CHUNK = 128

_ROWS_PER_TILE = _NP // _NS


def _sc_gather(table, idx):
    d = table.shape[1]
    per_w = _EP // _NW
    n_chunks = per_w // _CHUNK
    mesh = plsc.VectorSubcoreMesh(core_axis_name="c", subcore_axis_name="s")

    @functools.partial(
        pl.kernel, mesh=mesh,
        out_type=jax.ShapeDtypeStruct((_EP, d), jnp.float32),
        scratch_types=[
            pltpu.VMEM((_CHUNK,), jnp.int32),
            pltpu.VMEM((_CHUNK, d), jnp.float32),
            pltpu.SemaphoreType.DMA,
        ],
    )
    def k(table_hbm, idx_hbm, out_hbm, idx_v, rows_v, sem):
        wid = lax.axis_index("s") * _NC + lax.axis_index("c")
        base = wid * per_w

        def body(i, carry):
            off = base + i * _CHUNK
            pltpu.sync_copy(idx_hbm.at[pl.ds(off, _CHUNK)], idx_v)
            pltpu.async_copy(table_hbm.at[idx_v], rows_v, sem).wait()
            pltpu.sync_copy(rows_v, out_hbm.at[pl.ds(off, _CHUNK)])
            return carry

        lax.fori_loop(0, n_chunks, body, 0)

    return k(table, idx)


def _sc_scatter_add(msg, idx, zeros_nd):
    d = msg.shape[1]
    per_w = _EP // _NW
    n_chunks = per_w // _CHUNK
    mesh = plsc.VectorSubcoreMesh(core_axis_name="c", subcore_axis_name="s")

    @functools.partial(
        pl.kernel, mesh=mesh,
        out_type=jax.ShapeDtypeStruct((_NC, _NP, d), jnp.float32),
        scratch_types=[
            pltpu.VMEM((_CHUNK,), jnp.int32),
            pltpu.VMEM((_CHUNK, d), jnp.float32),
            pltpu.VMEM_SHARED((_NP, d), jnp.float32),
        ],
    )
    def k(msg_hbm, idx_hbm, zero_hbm, out_hbm, idx_v, chunk_v, acc_sh):
        cid = lax.axis_index("c")
        sid = lax.axis_index("s")
        wid = sid * _NC + cid
        r0 = sid * _ROWS_PER_TILE
        pltpu.sync_copy(zero_hbm.at[pl.ds(r0, _ROWS_PER_TILE)],
                        acc_sh.at[pl.ds(r0, _ROWS_PER_TILE)])
        plsc.subcore_barrier()
        base = wid * per_w

        def body(i, carry):
            off = base + i * _CHUNK
            pltpu.sync_copy(idx_hbm.at[pl.ds(off, _CHUNK)], idx_v)
            pltpu.sync_copy(msg_hbm.at[pl.ds(off, _CHUNK)], chunk_v)
            pltpu.sync_copy(chunk_v, acc_sh.at[idx_v], add=True)
            return carry

        lax.fori_loop(0, n_chunks, body, 0)
        plsc.subcore_barrier()
        pltpu.sync_copy(acc_sh.at[pl.ds(r0, _ROWS_PER_TILE)],
                        out_hbm.at[cid, pl.ds(r0, _ROWS_PER_TILE)])

    return k(msg, idx, zeros_nd)


def _ln(h, g, b):
    m = jnp.mean(h, axis=-1, keepdims=True)
    v = jnp.mean((h - m) ** 2, axis=-1, keepdims=True)
    return (h - m) * lax.rsqrt(v + 1e-5) * g + b


def _full2d(arr):
    return pl.BlockSpec(arr.shape, lambda i: (0, 0))


_NODE_T = 1024


def _rowspec(width, tile=_NODE_T):
    return pl.BlockSpec((tile, width), lambda i: (i, 0))


def _embed(xp, w0, b0, g0, be0, w1, b1, g1, be1):
    def body(x_ref, w0r, b0r, g0r, be0r, w1r, b1r, g1r, be1r, out_ref):
        h = jnp.maximum(x_ref[...] @ w0r[...] + b0r[...], 0.0)
        h = _ln(h, g0r[...], be0r[...])
        h = jnp.maximum(h @ w1r[...] + b1r[...], 0.0)
        out_ref[...] = _ln(h, g1r[...], be1r[...])

    ws = [w0, b0, g0, be0, w1, b1, g1, be1]
    return pl.pallas_call(
        body,
        grid=(_NP // _NODE_T,),
        in_specs=[_rowspec(9)] + [_full2d(w) for w in ws],
        out_specs=_rowspec(64),
        out_shape=jax.ShapeDtypeStruct((_NP, 64), jnp.float32),
    )(xp, *ws)


_EDGE_T = 512


def _msg(ea, xs, w1, b1, w2p, b2r, sel, out_c, in_c, out_w):
    xs_w = xs.shape[1]
    width = 32 * out_c
    n128 = width // 128
    g = 128 // out_c

    def body(ea_ref, xs_ref, w1r, b1r, w2pr, b2rr, selr, out_ref):
        hh = jnp.maximum(ea_ref[...] @ w1r[...] + b1r[...], 0.0)
        hhb = hh @ selr[...]
        xsv = xs_ref[...][:, :in_c]
        p = xsv @ w2pr[...]
        s = hhb[:, 0:128] * p[:, 0:128]
        for c in range(1, n128):
            s = s + hhb[:, c * 128:(c + 1) * 128] * p[:, c * 128:(c + 1) * 128]
        acc = xsv @ b2rr[...]
        for j in range(g):
            acc = acc + s[:, j * out_c:(j + 1) * out_c]
        if out_w > out_c:
            acc = jnp.concatenate(
                [acc, jnp.zeros((acc.shape[0], out_w - out_c), jnp.float32)],
                axis=1)
        out_ref[...] = acc

    ws = [w1, b1, w2p, b2r, sel]
    return pl.pallas_call(
        body,
        grid=(_EP // _EDGE_T,),
        in_specs=[_rowspec(8, _EDGE_T), _rowspec(xs_w, _EDGE_T)]
        + [_full2d(w) for w in ws],
        out_specs=_rowspec(out_w, _EDGE_T),
        out_shape=jax.ShapeDtypeStruct((_EP, out_w), jnp.float32),
    )(ea, xs, *ws)


def _node_update(h, a0, a1, root, bias, g, be):
    in_c = h.shape[1]
    out_c = root.shape[1]
    agg_w = a0.shape[1]

    def body(h_ref, a0_ref, a1_ref, rootr, biasr, gr, ber, out_ref):
        aggr = a0_ref[...][:, :out_c] + a1_ref[...][:, :out_c]
        s = aggr + h_ref[...] @ rootr[...] + biasr[...]
        out_ref[...] = _ln(jnp.maximum(s, 0.0), gr[...], ber[...])

    ws = [root, bias, g, be]
    return pl.pallas_call(
        body,
        grid=(_NP // _NODE_T,),
        in_specs=[_rowspec(in_c), _rowspec(agg_w), _rowspec(agg_w)]
        + [_full2d(w) for w in ws],
        out_specs=_rowspec(out_c),
        out_shape=jax.ShapeDtypeStruct((_NP, out_c), jnp.float32),
    )(h, a0, a1, *ws)


def _qkv_proj(h, wqkv, bqkv):
    def body(h_ref, wr, br, out_ref):
        out_ref[...] = h_ref[...] @ wr[...] + br[...]

    return pl.pallas_call(
        body,
        grid=(_NP // _NODE_T,),
        in_specs=[_rowspec(32), _full2d(wqkv), _full2d(bqkv)],
        out_specs=_rowspec(96),
        out_shape=jax.ShapeDtypeStruct((_NP, 96), jnp.float32),
    )(h, wqkv, bqkv)


_ATT_TQ = 256
_ATT_TK = 1024
_NHEAD = 4
_HD = 8


def _enc_layer(h, qkv, wo, bo, g1, be1, w1, b1, w2, b2, g2, be2):
    scale = 1.0 / (_HD ** 0.5)
    n_kt = _NP // _ATT_TK

    def body(h_ref, qkv_ref, wor, bor, g1r, be1r, w1r, b1r, w2r, b2r,
             g2r, be2r, out_ref):
        i = pl.program_id(0)
        q_all = qkv_ref[pl.ds(i * _ATT_TQ, _ATT_TQ), 0:32] * scale
        outs = []
        for hd in range(_NHEAD):
            qh = q_all[:, hd * _HD:(hd + 1) * _HD]

            def kv_step(j, carry, hd=hd, qh=qh):
                m, l, acc = carry
                kv = qkv_ref[pl.ds(j * _ATT_TK, _ATT_TK), :]
                kh = kv[:, 32 + hd * _HD:32 + (hd + 1) * _HD]
                vh = kv[:, 64 + hd * _HD:64 + (hd + 1) * _HD]
                s = lax.dot_general(qh, kh, (((1,), (1,)), ((), ())))
                kidx = j * _ATT_TK + lax.broadcasted_iota(
                    jnp.int32, (_ATT_TQ, _ATT_TK), 1)
                s = jnp.where(kidx >= _N, -1e30, s)
                m_new = jnp.maximum(m, jnp.max(s, axis=-1, keepdims=True))
                pexp = jnp.exp(s - m_new)
                corr = jnp.exp(m - m_new)
                l_new = l * corr + jnp.sum(pexp, axis=-1, keepdims=True)
                acc_new = acc * corr + pexp @ vh
                return m_new, l_new, acc_new

            m0 = jnp.full((_ATT_TQ, 1), -1e30, jnp.float32)
            l0 = jnp.zeros((_ATT_TQ, 1), jnp.float32)
            a0 = jnp.zeros((_ATT_TQ, _HD), jnp.float32)
            m, l, acc = lax.fori_loop(0, n_kt, kv_step, (m0, l0, a0))
            outs.append(acc / l)
        o = jnp.concatenate(outs, axis=-1)
        attn = o @ wor[...] + bor[...]
        x1 = _ln(h_ref[...] + attn, g1r[...], be1r[...])
        f = jnp.maximum(x1 @ w1r[...] + b1r[...], 0.0) @ w2r[...] + b2r[...]
        out_ref[...] = _ln(x1 + f, g2r[...], be2r[...])

    ws = [wo, bo, g1, be1, w1, b1, w2, b2, g2, be2]
    return pl.pallas_call(
        body,
        grid=(_NP // _ATT_TQ,),
        in_specs=[_rowspec(32, _ATT_TQ), _full2d(qkv)]
        + [_full2d(w) for w in ws],
        out_specs=_rowspec(32, _ATT_TQ),
        out_shape=jax.ShapeDtypeStruct((_NP, 32), jnp.float32),
    )(h, qkv, *ws)


def _head(h, fc_w, fc_b, out_w, out_b):
    def body(h_ref, fwr, fbr, owr, obr, out_ref):
        f = jnp.maximum(h_ref[...] @ fwr[...] + fbr[...], 0.0)
        out_ref[...] = f @ owr[...] + obr[...]

    ws = [fc_w, fc_b, out_w, out_b]
    return pl.pallas_call(
        body,
        grid=(_NP // _NODE_T,),
        in_specs=[_rowspec(32)] + [_full2d(w) for w in ws],
        out_specs=_rowspec(7),
        out_shape=jax.ShapeDtypeStruct((_NP, 7), jnp.float32),
    )(h, *ws)


def _r2(v):
    return v.reshape(1, -1)


def kernel(x, edge_index, edge_attr, params):
    p = params
    f32 = jnp.float32
    src = edge_index[0]
    dst = edge_index[1]

    xp = jnp.zeros((_NP, 9), f32).at[:_N].set(x)
    eap = jnp.zeros((_EP, 8), f32).at[:_E].set(edge_attr)
    srcp = jnp.zeros((_EP,), jnp.int32).at[:_E].set(src)
    dstp = jnp.full((_EP,), _N, jnp.int32).at[:_E].set(dst)
    z128 = jnp.zeros((_NP, 128), f32)

    w2p1 = p['c1_w2'].reshape(32, 64, 128).transpose(1, 0, 2).reshape(64, 32 * 128)
    b2r1 = p['c1_b2'].reshape(64, 128)
    w2p2 = p['c2_w2'].reshape(32, 128, 32).transpose(1, 0, 2).reshape(128, 32 * 32)
    b2r2 = p['c2_b2'].reshape(128, 32)
    sel1 = jnp.kron(jnp.eye(32, dtype=f32), jnp.ones((1, 128), f32))
    sel2 = jnp.kron(jnp.eye(32, dtype=f32), jnp.ones((1, 32), f32))

    h64 = _embed(xp, p['emb_w0'], _r2(p['emb_b0']), _r2(p['emb_g0']),
                 _r2(p['emb_be0']), p['emb_w1'], _r2(p['emb_b1']),
                 _r2(p['emb_g1']), _r2(p['emb_be1']))

    h64p = jnp.concatenate([h64, jnp.zeros((_NP, 64), f32)], axis=1)
    xs1 = _sc_gather(h64p, srcp)
    msg1 = _msg(eap, xs1, p['c1_w1'], _r2(p['c1_b1']), w2p1, b2r1, sel1,
                128, 64, 128)
    agg1 = _sc_scatter_add(msg1, dstp, z128)
    h128 = _node_update(h64, agg1[0], agg1[1], p['c1_root'],
                        _r2(p['c1_bias']), _r2(p['n1_g']), _r2(p['n1_be']))

    xs2 = _sc_gather(h128, srcp)
    msg2 = _msg(eap, xs2, p['c2_w1'], _r2(p['c2_b1']), w2p2, b2r2, sel2,
                32, 128, 128)
    agg2 = _sc_scatter_add(msg2, dstp, z128)
    h32 = _node_update(h128, agg2[0], agg2[1], p['c2_root'],
                       _r2(p['c2_bias']), _r2(p['n2_g']), _r2(p['n2_be']))

    for lp in p['layers']:
        qkv = _qkv_proj(h32, lp['wqkv'], _r2(lp['bqkv']))
        h32 = _enc_layer(h32, qkv, lp['wo'], _r2(lp['bo']), _r2(lp['g1']),
                         _r2(lp['be1']), lp['w1'], _r2(lp['b1']), lp['w2'],
                         _r2(lp['b2']), _r2(lp['g2']), _r2(lp['be2']))

    out = _head(h32, p['fc_w'], _r2(p['fc_b']), p['out_w'], _r2(p['out_b']))
    return out[:_N]


# speedup vs baseline: 1.2566x; 1.0168x over previous
import functools

import jax
import jax.numpy as jnp
from jax import lax
from jax.experimental import pallas as pl
from jax.experimental.pallas import tpu as pltpu
from jax.experimental.pallas import tpu_sc as plsc

_N = 10000
_E = 160000
_NP = 10240
_EP = 163840
_NC = 2
_NS = 16
_NW = _NC * _NS
_CHUNK = 128

_ROWS_PER_TILE = _NP // _NS


_GBATCH = 512


def _sc_gather(table, idx):
    d = table.shape[1]
    per_w = _EP // _NW
    n_b = per_w // _GBATCH
    n_fire = _GBATCH // _CHUNK
    mesh = plsc.VectorSubcoreMesh(core_axis_name="c", subcore_axis_name="s")

    @functools.partial(
        pl.kernel, mesh=mesh,
        out_type=jax.ShapeDtypeStruct((_EP, d), jnp.float32),
        scratch_types=[
            pltpu.VMEM((_GBATCH,), jnp.int32),
            pltpu.VMEM((_GBATCH, d), jnp.float32),
            pltpu.SemaphoreType.DMA,
        ],
    )
    def k(table_hbm, idx_hbm, out_hbm, idx_v, rows_v, sem):
        wid = lax.axis_index("s") * _NC + lax.axis_index("c")
        base = wid * per_w

        def body(i, carry):
            off = base + i * _GBATCH
            pltpu.sync_copy(idx_hbm.at[pl.ds(off, _GBATCH)], idx_v)
            cps = [
                pltpu.async_copy(
                    table_hbm.at[idx_v.at[pl.ds(j * _CHUNK, _CHUNK)]],
                    rows_v.at[pl.ds(j * _CHUNK, _CHUNK)], sem)
                for j in range(n_fire)
            ]
            for c in cps:
                c.wait()
            pltpu.sync_copy(rows_v, out_hbm.at[pl.ds(off, _GBATCH)])
            return carry

        lax.fori_loop(0, n_b, body, 0)

    return k(table, idx)


def _sc_scatter_add(msg, idx, zeros_nd):
    d = msg.shape[1]
    per_w = _EP // _NW
    n_chunks = per_w // _CHUNK
    mesh = plsc.VectorSubcoreMesh(core_axis_name="c", subcore_axis_name="s")

    @functools.partial(
        pl.kernel, mesh=mesh,
        out_type=jax.ShapeDtypeStruct((_NC, _NP, d), jnp.float32),
        scratch_types=[
            pltpu.VMEM((_CHUNK,), jnp.int32),
            pltpu.VMEM((_CHUNK, d), jnp.float32),
            pltpu.VMEM_SHARED((_NP, d), jnp.float32),
        ],
    )
    def k(msg_hbm, idx_hbm, zero_hbm, out_hbm, idx_v, chunk_v, acc_sh):
        cid = lax.axis_index("c")
        sid = lax.axis_index("s")
        wid = sid * _NC + cid
        r0 = sid * _ROWS_PER_TILE
        pltpu.sync_copy(zero_hbm.at[pl.ds(r0, _ROWS_PER_TILE)],
                        acc_sh.at[pl.ds(r0, _ROWS_PER_TILE)])
        plsc.subcore_barrier()
        base = wid * per_w

        def body(i, carry):
            off = base + i * _CHUNK
            pltpu.sync_copy(idx_hbm.at[pl.ds(off, _CHUNK)], idx_v)
            pltpu.sync_copy(msg_hbm.at[pl.ds(off, _CHUNK)], chunk_v)
            pltpu.sync_copy(chunk_v, acc_sh.at[idx_v], add=True)
            return carry

        lax.fori_loop(0, n_chunks, body, 0)
        plsc.subcore_barrier()
        pltpu.sync_copy(acc_sh.at[pl.ds(r0, _ROWS_PER_TILE)],
                        out_hbm.at[cid, pl.ds(r0, _ROWS_PER_TILE)])

    return k(msg, idx, zeros_nd)


def _ln(h, g, b):
    m = jnp.mean(h, axis=-1, keepdims=True)
    v = jnp.mean((h - m) ** 2, axis=-1, keepdims=True)
    return (h - m) * lax.rsqrt(v + 1e-5) * g + b


def _full2d(arr):
    return pl.BlockSpec(arr.shape, lambda i: (0, 0))


_NODE_T = 1024


def _rowspec(width, tile=_NODE_T):
    return pl.BlockSpec((tile, width), lambda i: (i, 0))


def _embed(xp, w0, b0, g0, be0, w1, b1, g1, be1):
    def body(x_ref, w0r, b0r, g0r, be0r, w1r, b1r, g1r, be1r, out_ref):
        h = jnp.maximum(x_ref[...] @ w0r[...] + b0r[...], 0.0)
        h = _ln(h, g0r[...], be0r[...])
        h = jnp.maximum(h @ w1r[...] + b1r[...], 0.0)
        out_ref[...] = _ln(h, g1r[...], be1r[...])

    ws = [w0, b0, g0, be0, w1, b1, g1, be1]
    return pl.pallas_call(
        body,
        grid=(_NP // _NODE_T,),
        in_specs=[_rowspec(9)] + [_full2d(w) for w in ws],
        out_specs=_rowspec(64),
        out_shape=jax.ShapeDtypeStruct((_NP, 64), jnp.float32),
    )(xp, *ws)


_EDGE_T = 512


def _msg(ea, xs, w1, b1, w2p, b2r, sel, out_c, in_c, out_w):
    xs_w = xs.shape[1]
    width = 32 * out_c
    n128 = width // 128
    g = 128 // out_c

    def body(ea_ref, xs_ref, w1r, b1r, w2pr, b2rr, selr, out_ref):
        hh = jnp.maximum(ea_ref[...] @ w1r[...] + b1r[...], 0.0)
        hhb = hh @ selr[...]
        xsv = xs_ref[...][:, :in_c]
        p = xsv @ w2pr[...]
        s = hhb[:, 0:128] * p[:, 0:128]
        for c in range(1, n128):
            s = s + hhb[:, c * 128:(c + 1) * 128] * p[:, c * 128:(c + 1) * 128]
        acc = xsv @ b2rr[...]
        for j in range(g):
            acc = acc + s[:, j * out_c:(j + 1) * out_c]
        if out_w > out_c:
            acc = jnp.concatenate(
                [acc, jnp.zeros((acc.shape[0], out_w - out_c), jnp.float32)],
                axis=1)
        out_ref[...] = acc

    ws = [w1, b1, w2p, b2r, sel]
    return pl.pallas_call(
        body,
        grid=(_EP // _EDGE_T,),
        in_specs=[_rowspec(8, _EDGE_T), _rowspec(xs_w, _EDGE_T)]
        + [_full2d(w) for w in ws],
        out_specs=_rowspec(out_w, _EDGE_T),
        out_shape=jax.ShapeDtypeStruct((_EP, out_w), jnp.float32),
    )(ea, xs, *ws)


def _node_update(h, a0, a1, root, bias, g, be):
    in_c = h.shape[1]
    out_c = root.shape[1]
    agg_w = a0.shape[1]

    def body(h_ref, a0_ref, a1_ref, rootr, biasr, gr, ber, out_ref):
        aggr = a0_ref[...][:, :out_c] + a1_ref[...][:, :out_c]
        s = aggr + h_ref[...] @ rootr[...] + biasr[...]
        out_ref[...] = _ln(jnp.maximum(s, 0.0), gr[...], ber[...])

    ws = [root, bias, g, be]
    return pl.pallas_call(
        body,
        grid=(_NP // _NODE_T,),
        in_specs=[_rowspec(in_c), _rowspec(agg_w), _rowspec(agg_w)]
        + [_full2d(w) for w in ws],
        out_specs=_rowspec(out_c),
        out_shape=jax.ShapeDtypeStruct((_NP, out_c), jnp.float32),
    )(h, a0, a1, *ws)


def _qkv_proj(h, wqkv, bqkv):
    def body(h_ref, wr, br, out_ref):
        out_ref[...] = h_ref[...] @ wr[...] + br[...]

    return pl.pallas_call(
        body,
        grid=(_NP // _NODE_T,),
        in_specs=[_rowspec(32), _full2d(wqkv), _full2d(bqkv)],
        out_specs=_rowspec(96),
        out_shape=jax.ShapeDtypeStruct((_NP, 96), jnp.float32),
    )(h, wqkv, bqkv)


_ATT_TQ = 256
_ATT_TK = 1024
_NHEAD = 4
_HD = 8


def _enc_layer(h, qkv, wo, bo, g1, be1, w1, b1, w2, b2, g2, be2):
    scale = 1.0 / (_HD ** 0.5)
    n_kt = _NP // _ATT_TK

    def body(h_ref, qkv_ref, wor, bor, g1r, be1r, w1r, b1r, w2r, b2r,
             g2r, be2r, out_ref):
        i = pl.program_id(0)
        q_all = qkv_ref[pl.ds(i * _ATT_TQ, _ATT_TQ), 0:32] * scale
        outs = []
        for hd in range(_NHEAD):
            qh = q_all[:, hd * _HD:(hd + 1) * _HD]

            def kv_step(j, carry, hd=hd, qh=qh):
                m, l, acc = carry
                kv = qkv_ref[pl.ds(j * _ATT_TK, _ATT_TK), :]
                kh = kv[:, 32 + hd * _HD:32 + (hd + 1) * _HD]
                vh = kv[:, 64 + hd * _HD:64 + (hd + 1) * _HD]
                s = lax.dot_general(qh, kh, (((1,), (1,)), ((), ())))
                kidx = j * _ATT_TK + lax.broadcasted_iota(
                    jnp.int32, (_ATT_TQ, _ATT_TK), 1)
                s = jnp.where(kidx >= _N, -1e30, s)
                m_new = jnp.maximum(m, jnp.max(s, axis=-1, keepdims=True))
                pexp = jnp.exp(s - m_new)
                corr = jnp.exp(m - m_new)
                l_new = l * corr + jnp.sum(pexp, axis=-1, keepdims=True)
                acc_new = acc * corr + pexp @ vh
                return m_new, l_new, acc_new

            m0 = jnp.full((_ATT_TQ, 1), -1e30, jnp.float32)
            l0 = jnp.zeros((_ATT_TQ, 1), jnp.float32)
            a0 = jnp.zeros((_ATT_TQ, _HD), jnp.float32)
            m, l, acc = lax.fori_loop(0, n_kt, kv_step, (m0, l0, a0))
            outs.append(acc / l)
        o = jnp.concatenate(outs, axis=-1)
        attn = o @ wor[...] + bor[...]
        x1 = _ln(h_ref[...] + attn, g1r[...], be1r[...])
        f = jnp.maximum(x1 @ w1r[...] + b1r[...], 0.0) @ w2r[...] + b2r[...]
        out_ref[...] = _ln(x1 + f, g2r[...], be2r[...])

    ws = [wo, bo, g1, be1, w1, b1, w2, b2, g2, be2]
    return pl.pallas_call(
        body,
        grid=(_NP // _ATT_TQ,),
        in_specs=[_rowspec(32, _ATT_TQ), _full2d(qkv)]
        + [_full2d(w) for w in ws],
        out_specs=_rowspec(32, _ATT_TQ),
        out_shape=jax.ShapeDtypeStruct((_NP, 32), jnp.float32),
    )(h, qkv, *ws)


def _head(h, fc_w, fc_b, out_w, out_b):
    def body(h_ref, fwr, fbr, owr, obr, out_ref):
        f = jnp.maximum(h_ref[...] @ fwr[...] + fbr[...], 0.0)
        out_ref[...] = f @ owr[...] + obr[...]

    ws = [fc_w, fc_b, out_w, out_b]
    return pl.pallas_call(
        body,
        grid=(_NP // _NODE_T,),
        in_specs=[_rowspec(32)] + [_full2d(w) for w in ws],
        out_specs=_rowspec(7),
        out_shape=jax.ShapeDtypeStruct((_NP, 7), jnp.float32),
    )(h, *ws)


def _r2(v):
    return v.reshape(1, -1)


def kernel(x, edge_index, edge_attr, params):
    p = params
    f32 = jnp.float32
    src = edge_index[0]
    dst = edge_index[1]

    xp = jnp.zeros((_NP, 9), f32).at[:_N].set(x)
    eap = jnp.zeros((_EP, 8), f32).at[:_E].set(edge_attr)
    srcp = jnp.zeros((_EP,), jnp.int32).at[:_E].set(src)
    dstp = jnp.full((_EP,), _N, jnp.int32).at[:_E].set(dst)
    z128 = jnp.zeros((_NP, 128), f32)

    w2p1 = p['c1_w2'].reshape(32, 64, 128).transpose(1, 0, 2).reshape(64, 32 * 128)
    b2r1 = p['c1_b2'].reshape(64, 128)
    w2p2 = p['c2_w2'].reshape(32, 128, 32).transpose(1, 0, 2).reshape(128, 32 * 32)
    b2r2 = p['c2_b2'].reshape(128, 32)
    sel1 = jnp.kron(jnp.eye(32, dtype=f32), jnp.ones((1, 128), f32))
    sel2 = jnp.kron(jnp.eye(32, dtype=f32), jnp.ones((1, 32), f32))

    h64 = _embed(xp, p['emb_w0'], _r2(p['emb_b0']), _r2(p['emb_g0']),
                 _r2(p['emb_be0']), p['emb_w1'], _r2(p['emb_b1']),
                 _r2(p['emb_g1']), _r2(p['emb_be1']))

    h64p = jnp.concatenate([h64, jnp.zeros((_NP, 64), f32)], axis=1)
    xs1 = _sc_gather(h64p, srcp)
    msg1 = _msg(eap, xs1, p['c1_w1'], _r2(p['c1_b1']), w2p1, b2r1, sel1,
                128, 64, 128)
    agg1 = _sc_scatter_add(msg1, dstp, z128)
    h128 = _node_update(h64, agg1[0], agg1[1], p['c1_root'],
                        _r2(p['c1_bias']), _r2(p['n1_g']), _r2(p['n1_be']))

    xs2 = _sc_gather(h128, srcp)
    msg2 = _msg(eap, xs2, p['c2_w1'], _r2(p['c2_b1']), w2p2, b2r2, sel2,
                32, 128, 128)
    agg2 = _sc_scatter_add(msg2, dstp, z128)
    h32 = _node_update(h128, agg2[0], agg2[1], p['c2_root'],
                       _r2(p['c2_bias']), _r2(p['n2_g']), _r2(p['n2_be']))

    for lp in p['layers']:
        qkv = _qkv_proj(h32, lp['wqkv'], _r2(lp['bqkv']))
        h32 = _enc_layer(h32, qkv, lp['wo'], _r2(lp['bo']), _r2(lp['g1']),
                         _r2(lp['be1']), lp['w1'], _r2(lp['b1']), lp['w2'],
                         _r2(lp['b2']), _r2(lp['g2']), _r2(lp['be2']))

    out = _head(h32, p['fc_w'], _r2(p['fc_b']), p['out_w'], _r2(p['out_b']))
    return out[:_N]
